# Initial kernel scaffold; baseline (speedup 1.0000x reference)
import jax
import jax.numpy as jnp
from jax.experimental import pallas as pl


def kernel(at_no, embed_ten):
    raise NotImplementedError("write your pallas kernel here")


# speedup vs baseline: 1.2943x; 1.2943x over previous
import functools

import jax
import jax.numpy as jnp
from jax import lax
from jax.experimental import pallas as pl
from jax.experimental.pallas import tpu as pltpu
from jax.experimental.pallas import tpu_sc as plsc

B = 100000
D = 256
C = 80
NC = 2
NS = 16
NW = NC * NS
NCHUNKS = B // C


def _body(at_no_hbm, table_hbm, out_hbm, idx_v, rows_v, sem):
    c = lax.axis_index("c")
    s = lax.axis_index("s")
    wid = s * NC + c
    nloc = (NCHUNKS - wid + NW - 1) // NW

    def step(i, carry):
        start = (wid + i * NW) * C
        pltpu.sync_copy(at_no_hbm.at[pl.ds(start, C)], idx_v)
        pltpu.async_copy(table_hbm.at[idx_v], rows_v, sem).wait()
        pltpu.sync_copy(rows_v, out_hbm.at[pl.ds(start, C)])
        return carry

    lax.fori_loop(0, nloc, step, 0)


def kernel(at_no, embed_ten):
    mesh = plsc.VectorSubcoreMesh(core_axis_name="c", subcore_axis_name="s")
    k = functools.partial(
        pl.kernel,
        mesh=mesh,
        out_type=jax.ShapeDtypeStruct((B, D), jnp.float32),
        scratch_types=[
            pltpu.VMEM((C,), jnp.int32),
            pltpu.VMEM((C, D), jnp.float32),
            pltpu.SemaphoreType.DMA,
        ],
    )(_body)
    return k(at_no, embed_ten)


# speedup vs baseline: 1.3057x; 1.0089x over previous
import functools

import jax
import jax.numpy as jnp
from jax import lax
from jax.experimental import pallas as pl
from jax.experimental.pallas import tpu as pltpu
from jax.experimental.pallas import tpu_sc as plsc

B = 100000
D = 256
C = 80
NC = 2
NS = 16
NW = NC * NS
NCHUNKS = B // C


NBUF = 2


def _body(at_no_hbm, table_hbm, out_hbm, idx_v, rows_v, sem_g, sem_s0, sem_s1):
    sem_s = (sem_s0, sem_s1)
    c = lax.axis_index("c")
    s = lax.axis_index("s")
    wid = s * NC + c
    nloc = (NCHUNKS - wid + NW - 1) // NW

    def group(g, carry):
        for b in range(NBUF):
            i = g * NBUF + b
            start = (wid + i * NW) * C

            @pl.when(i < nloc)
            def _():
                @pl.when(i >= NBUF)
                def _():
                    pltpu.make_async_copy(
                        rows_v.at[b], out_hbm.at[pl.ds(0, C)], sem_s[b]
                    ).wait()

                pltpu.sync_copy(at_no_hbm.at[pl.ds(start, C)], idx_v.at[b])
                pltpu.async_copy(table_hbm.at[idx_v.at[b]], rows_v.at[b], sem_g).wait()
                pltpu.async_copy(rows_v.at[b], out_hbm.at[pl.ds(start, C)], sem_s[b])
        return carry

    ngroups = (nloc + NBUF - 1) // NBUF
    lax.fori_loop(0, ngroups, group, 0)

    for b in range(NBUF):
        pltpu.make_async_copy(rows_v.at[b], out_hbm.at[pl.ds(0, C)], sem_s[b]).wait()


def kernel(at_no, embed_ten):
    mesh = plsc.VectorSubcoreMesh(core_axis_name="c", subcore_axis_name="s")
    k = functools.partial(
        pl.kernel,
        mesh=mesh,
        out_type=jax.ShapeDtypeStruct((B, D), jnp.float32),
        scratch_types=[
            pltpu.VMEM((NBUF, C), jnp.int32),
            pltpu.VMEM((NBUF, C, D), jnp.float32),
            pltpu.SemaphoreType.DMA,
            pltpu.SemaphoreType.DMA,
            pltpu.SemaphoreType.DMA,
        ],
    )(_body)
    return k(at_no, embed_ten)
